# Initial kernel scaffold; baseline (speedup 1.0000x reference)
import jax
import jax.numpy as jnp
from jax.experimental import pallas as pl


def kernel(x, edge_index, edge_type, emb, W, W_root, b, ln_w, ln_b):
    raise NotImplementedError("write your pallas kernel here")


# speedup vs baseline: 10.6938x; 10.6938x over previous
import functools

import jax
import jax.numpy as jnp
from jax import lax
from jax.experimental import pallas as pl
from jax.experimental.pallas import tpu as pltpu
from jax.experimental.pallas import tpu_sc as plsc

N = 10000
E = 320000
D = 128
R = 8
L_LAYERS = 3
EPS = 1e-5

NC = 2
NS = 16
NW = NC * NS
LANES = 16

K = 80
EPW = E // NW
NBLK = EPW // K
EPS_TILE = E // NS
NBLK_CNT = EPS_TILE // K
ROWS_PT = N // NS
RBLK = (N // K + NW - 1) // NW

_mesh = plsc.VectorSubcoreMesh(core_axis_name="c", subcore_axis_name="s",
                               num_cores=NC, num_subcores=NS)


def _f32(shape):
    return jax.ShapeDtypeStruct(shape, jnp.float32)


def _i32(shape):
    return jax.ShapeDtypeStruct(shape, jnp.int32)


@functools.partial(
    pl.kernel,
    out_type=(_i32((E,)), _f32((E,)), _f32((N, D))),
    mesh=_mesh,
    compiler_params=pltpu.CompilerParams(needs_layout_passes=False),
    scratch_types=[
        pltpu.VMEM_SHARED((N * R,), jnp.float32),
        pltpu.VMEM((K,), jnp.int32),
        pltpu.VMEM((K,), jnp.int32),
        pltpu.VMEM((K,), jnp.int32),
        pltpu.VMEM((K,), jnp.int32),
        pltpu.VMEM((K,), jnp.int32),
        pltpu.VMEM((K,), jnp.float32),
        pltpu.VMEM((K,), jnp.float32),
        pltpu.VMEM((K,), jnp.float32),
        pltpu.VMEM((K,), jnp.int32),
        pltpu.VMEM((K, D), jnp.float32),
        pltpu.VMEM(((N * R) // NS,), jnp.float32),
        pltpu.SemaphoreType.DMA,
    ],
)
def _prep_kernel(src_hbm, dst_hbm, ety_hbm, x_hbm, emb_hbm, zcnt_hbm,
                 g_hbm, scale_hbm, h0_hbm,
                 cnt_sp, dbuf, tbuf, sbufi, cidx, gbuf, ones, cval, sbuf,
                 xbuf, rows, zstage, sem):
    c = lax.axis_index("c")
    s = lax.axis_index("s")
    wid = s * NC + c

    cslice = (N * R) // NS
    pltpu.sync_copy(zcnt_hbm.at[pl.ds(s * cslice, cslice)], zstage)
    pltpu.sync_copy(zstage, cnt_sp.at[pl.ds(s * cslice, cslice)])
    for i in range(K // LANES):
        ones[pl.ds(i * LANES, LANES)] = jnp.ones((LANES,), jnp.float32)
    plsc.subcore_barrier()

    @pl.loop(0, NBLK_CNT)
    def _count(j):
        off = s * EPS_TILE + j * K
        pltpu.sync_copy(dst_hbm.at[pl.ds(off, K)], dbuf)
        pltpu.sync_copy(ety_hbm.at[pl.ds(off, K)], tbuf)
        for i in range(K // LANES):
            sl = pl.ds(i * LANES, LANES)
            cidx[sl] = dbuf[sl] * R + tbuf[sl]
        pltpu.sync_copy(ones, cnt_sp.at[cidx], add=True)

    plsc.subcore_barrier()

    @pl.loop(0, NBLK)
    def _scale(j):
        off = wid * EPW + j * K
        pltpu.sync_copy(dst_hbm.at[pl.ds(off, K)], dbuf)
        pltpu.sync_copy(ety_hbm.at[pl.ds(off, K)], tbuf)
        pltpu.sync_copy(src_hbm.at[pl.ds(off, K)], sbufi)
        for i in range(K // LANES):
            sl = pl.ds(i * LANES, LANES)
            t16 = tbuf[sl]
            cidx[sl] = dbuf[sl] * R + t16
            gbuf[sl] = sbufi[sl] * R + t16
        pltpu.sync_copy(gbuf, g_hbm.at[pl.ds(off, K)])
        pltpu.async_copy(cnt_sp.at[cidx], cval, sem).wait()
        for i in range(K // LANES):
            sl = pl.ds(i * LANES, LANES)
            sbuf[sl] = 1.0 / jnp.maximum(cval[sl], 1.0)
        pltpu.sync_copy(sbuf, scale_hbm.at[pl.ds(off, K)])

    for jj in range(RBLK):
        blk = wid + jj * NW

        @pl.when(blk < N // K)
        def _():
            off = blk * K
            pltpu.sync_copy(x_hbm.at[pl.ds(off, K)], xbuf)
            pltpu.async_copy(emb_hbm.at[xbuf], rows, sem).wait()
            pltpu.sync_copy(rows, h0_hbm.at[pl.ds(off, K)])


@functools.partial(
    pl.kernel,
    out_type=_f32((NC * N, D)),
    mesh=_mesh,
    compiler_params=pltpu.CompilerParams(needs_layout_passes=False),
    scratch_types=[
        pltpu.VMEM_SHARED((N, D), jnp.float32),
        pltpu.VMEM((K,), jnp.int32),
        pltpu.VMEM((K,), jnp.int32),
        pltpu.VMEM((K + LANES,), jnp.float32),
        pltpu.VMEM((K, D), jnp.float32),
        pltpu.SemaphoreType.DMA,
    ],
)
def _agg_kernel(t_hbm, g_hbm, dst_hbm, scale_hbm, zacc_hbm, acc_hbm,
                acc_sp, gbuf, dbuf, sbuf, rows, sem):
    c = lax.axis_index("c")
    s = lax.axis_index("s")
    wid = s * NC + c

    NCH = N // K
    for q in range((NCH + NS - 1) // NS):
        ch = s + q * NS

        @pl.when(ch < NCH)
        def _():
            off = ch * K
            pltpu.sync_copy(zacc_hbm.at[pl.ds(off, K)], rows)
            pltpu.sync_copy(rows, acc_sp.at[pl.ds(off, K)])

    plsc.subcore_barrier()

    @pl.loop(0, NBLK)
    def _blk(j):
        off = wid * EPW + j * K
        pltpu.sync_copy(g_hbm.at[pl.ds(off, K)], gbuf)
        pltpu.sync_copy(dst_hbm.at[pl.ds(off, K)], dbuf)
        pltpu.sync_copy(scale_hbm.at[pl.ds(off, K)], sbuf.at[pl.ds(LANES, K)])
        pltpu.async_copy(t_hbm.at[gbuf], rows, sem).wait()

        for k in range(K):
            bc = plsc.load_gather(
                sbuf, [jnp.full((LANES,), k + LANES, jnp.int32)])
            for j8 in range(D // LANES):
                sl = pl.ds(j8 * LANES, LANES)
                rows[k, sl] = rows[k, sl] * bc

        pltpu.sync_copy(rows, acc_sp.at[dbuf], add=True)

    plsc.subcore_barrier()
    for q in range((NCH + NS - 1) // NS):
        ch = s + q * NS

        @pl.when(ch < NCH)
        def _():
            off = ch * K
            pltpu.sync_copy(acc_sp.at[pl.ds(off, K)], rows)
            pltpu.sync_copy(rows, acc_hbm.at[pl.ds(c * N + off, K)])


BN = 400


def _mm_body(h_ref, w_ref, bf_ref, t_ref, o_ref):
    prod = jnp.dot(h_ref[...], w_ref[...], preferred_element_type=jnp.float32)
    prod = prod + bf_ref[...]
    t_ref[...] = prod[:, :R * D]
    o_ref[...] = prod[:, R * D:]


def _tc_matmul(h, bigw, bfull):
    return pl.pallas_call(
        _mm_body,
        grid=(N // BN,),
        in_specs=[
            pl.BlockSpec((BN, D), lambda i: (i, 0)),
            pl.BlockSpec((D, R * D + D), lambda i: (0, 0)),
            pl.BlockSpec((1, R * D + D), lambda i: (0, 0)),
        ],
        out_specs=[
            pl.BlockSpec((BN, R * D), lambda i: (i, 0)),
            pl.BlockSpec((BN, D), lambda i: (i, 0)),
        ],
        out_shape=[_f32((N, R * D)), _f32((N, D))],
    )(h, bigw, bfull)


def _make_ln_body(layer):
    def body(o_ref, a0_ref, a1_ref, hp_ref, w_ref, b_ref, out_ref):
        v = o_ref[...] + a0_ref[...] + a1_ref[...]
        mu = jnp.mean(v, axis=-1, keepdims=True)
        var = jnp.mean((v - mu) ** 2, axis=-1, keepdims=True)
        y = (v - mu) / jnp.sqrt(var + EPS) * w_ref[...] + b_ref[...]
        y = jnp.maximum(y, 0.0)
        if layer > 0:
            y = y + hp_ref[...]
        out_ref[...] = y
    return body


def _tc_ln(layer, out0, acc0, acc1, h_prev, lnw, lnb):
    return pl.pallas_call(
        _make_ln_body(layer),
        grid=(N // BN,),
        in_specs=[
            pl.BlockSpec((BN, D), lambda i: (i, 0)),
            pl.BlockSpec((BN, D), lambda i: (i, 0)),
            pl.BlockSpec((BN, D), lambda i: (i, 0)),
            pl.BlockSpec((BN, D), lambda i: (i, 0)),
            pl.BlockSpec((1, D), lambda i: (0, 0)),
            pl.BlockSpec((1, D), lambda i: (0, 0)),
        ],
        out_specs=pl.BlockSpec((BN, D), lambda i: (i, 0)),
        out_shape=_f32((N, D)),
    )(out0, acc0, acc1, h_prev, lnw, lnb)


def kernel(x, edge_index, edge_type, emb, W, W_root, b, ln_w, ln_b):
    src = edge_index[0].astype(jnp.int32)
    dst = edge_index[1].astype(jnp.int32)
    ety = edge_type.astype(jnp.int32)
    zcnt = jnp.zeros((N * R,), jnp.float32)
    zacc = jnp.zeros((N, D), jnp.float32)

    g, scale, h = _prep_kernel(src, dst, ety, x.astype(jnp.int32), emb, zcnt)

    for i in range(L_LAYERS):
        bigw = jnp.concatenate(
            [W[i].transpose(1, 0, 2).reshape(D, R * D), W_root[i]], axis=1)
        bfull = jnp.concatenate(
            [jnp.zeros((R * D,), jnp.float32), b[i]]).reshape(1, R * D + D)
        t2d, out0 = _tc_matmul(h, bigw, bfull)
        t = t2d.reshape(N * R, D)
        accf = _agg_kernel(t, g, dst, scale, zacc)
        h_new = _tc_ln(i, out0, accf[:N], accf[N:], h,
                       ln_w[i].reshape(1, D), ln_b[i].reshape(1, D))
        h = h_new
    return h


# speedup vs baseline: 17.3725x; 1.6245x over previous
import functools

import jax
import jax.numpy as jnp
from jax import lax
from jax.experimental import pallas as pl
from jax.experimental.pallas import tpu as pltpu
from jax.experimental.pallas import tpu_sc as plsc

N = 10000
E = 320000
D = 128
R = 8
L_LAYERS = 3
EPS = 1e-5

NC = 2
NS = 16
NW = NC * NS
LANES = 16

K = 80
EPW = E // NW
NBLK = EPW // K
NSB = 5
EPS_TILE = E // NS
NBLK_CNT = EPS_TILE // K
ROWS_PT = N // NS
RBLK = (N // K + NW - 1) // NW

_mesh = plsc.VectorSubcoreMesh(core_axis_name="c", subcore_axis_name="s",
                               num_cores=NC, num_subcores=NS)


def _f32(shape):
    return jax.ShapeDtypeStruct(shape, jnp.float32)


def _i32(shape):
    return jax.ShapeDtypeStruct(shape, jnp.int32)


@functools.partial(
    pl.kernel,
    out_type=(_i32((E,)), _f32((E,)), _f32((N, D))),
    mesh=_mesh,
    compiler_params=pltpu.CompilerParams(needs_layout_passes=False),
    scratch_types=[
        pltpu.VMEM_SHARED((N * R,), jnp.float32),
        pltpu.VMEM((K,), jnp.int32),
        pltpu.VMEM((K,), jnp.int32),
        pltpu.VMEM((K,), jnp.int32),
        pltpu.VMEM((K,), jnp.int32),
        pltpu.VMEM((K,), jnp.int32),
        pltpu.VMEM((K,), jnp.float32),
        pltpu.VMEM((K,), jnp.float32),
        pltpu.VMEM((K,), jnp.float32),
        pltpu.VMEM((K,), jnp.int32),
        pltpu.VMEM((K, D), jnp.float32),
        pltpu.VMEM(((N * R) // NS,), jnp.float32),
        pltpu.SemaphoreType.DMA,
    ],
)
def _prep_kernel(src_hbm, dst_hbm, ety_hbm, x_hbm, emb_hbm, zcnt_hbm,
                 g_hbm, scale_hbm, h0_hbm,
                 cnt_sp, dbuf, tbuf, sbufi, cidx, gbuf, ones, cval, sbuf,
                 xbuf, rows, zstage, sem):
    c = lax.axis_index("c")
    s = lax.axis_index("s")
    wid = s * NC + c

    cslice = (N * R) // NS
    pltpu.sync_copy(zcnt_hbm.at[pl.ds(s * cslice, cslice)], zstage)
    pltpu.sync_copy(zstage, cnt_sp.at[pl.ds(s * cslice, cslice)])
    for i in range(K // LANES):
        ones[pl.ds(i * LANES, LANES)] = jnp.ones((LANES,), jnp.float32)
    plsc.subcore_barrier()

    @pl.loop(0, NBLK_CNT)
    def _count(j):
        off = s * EPS_TILE + j * K
        pltpu.sync_copy(dst_hbm.at[pl.ds(off, K)], dbuf)
        pltpu.sync_copy(ety_hbm.at[pl.ds(off, K)], tbuf)
        for i in range(K // LANES):
            sl = pl.ds(i * LANES, LANES)
            cidx[sl] = dbuf[sl] * R + tbuf[sl]
        pltpu.sync_copy(ones, cnt_sp.at[cidx], add=True)

    plsc.subcore_barrier()

    @pl.loop(0, NBLK)
    def _scale(j):
        off = wid * EPW + j * K
        pltpu.sync_copy(dst_hbm.at[pl.ds(off, K)], dbuf)
        pltpu.sync_copy(ety_hbm.at[pl.ds(off, K)], tbuf)
        pltpu.sync_copy(src_hbm.at[pl.ds(off, K)], sbufi)
        for i in range(K // LANES):
            sl = pl.ds(i * LANES, LANES)
            t16 = tbuf[sl]
            cidx[sl] = dbuf[sl] * R + t16
            gbuf[sl] = sbufi[sl] * R + t16
        pltpu.sync_copy(gbuf, g_hbm.at[pl.ds(off, K)])
        pltpu.async_copy(cnt_sp.at[cidx], cval, sem).wait()
        for i in range(K // LANES):
            sl = pl.ds(i * LANES, LANES)
            sbuf[sl] = 1.0 / jnp.maximum(cval[sl], 1.0)
        pltpu.sync_copy(sbuf, scale_hbm.at[pl.ds(off, K)])

    for jj in range(RBLK):
        blk = wid + jj * NW

        @pl.when(blk < N // K)
        def _():
            off = blk * K
            pltpu.sync_copy(x_hbm.at[pl.ds(off, K)], xbuf)
            pltpu.async_copy(emb_hbm.at[xbuf], rows, sem).wait()
            pltpu.sync_copy(rows, h0_hbm.at[pl.ds(off, K)])


@functools.partial(
    pl.kernel,
    out_type=_f32((NC * N, D)),
    mesh=_mesh,
    compiler_params=pltpu.CompilerParams(needs_layout_passes=False),
    scratch_types=[
        pltpu.VMEM_SHARED((N, D), jnp.float32),
        pltpu.VMEM((NBLK // NSB, K), jnp.int32),
        pltpu.VMEM((NBLK // NSB, K), jnp.int32),
        pltpu.VMEM(((NBLK // NSB) * K + LANES,), jnp.float32),
        pltpu.VMEM((K, D), jnp.float32),
        pltpu.VMEM((K, D), jnp.float32),
        pltpu.SemaphoreType.DMA,
        pltpu.SemaphoreType.DMA,
    ],
)
def _agg_kernel(t_hbm, g_hbm, dst_hbm, scale_hbm, zacc_hbm, acc_hbm,
                acc_sp, gbuf, dbuf, sbuf, rows_a, rows_b, sem_a, sem_b):
    c = lax.axis_index("c")
    s = lax.axis_index("s")
    wid = s * NC + c

    NCH = N // K
    for q in range((NCH + NS - 1) // NS):
        ch = s + q * NS

        @pl.when(ch < NCH)
        def _():
            off = ch * K
            pltpu.sync_copy(zacc_hbm.at[pl.ds(off, K)], rows_a)
            pltpu.sync_copy(rows_a, acc_sp.at[pl.ds(off, K)])

    plsc.subcore_barrier()

    NB2 = NBLK // NSB

    def start_gather(j, rows, sem):
        pltpu.async_copy(t_hbm.at[gbuf.at[j]], rows, sem)

    def finish_block(j, rows, sem):
        pltpu.make_async_copy(t_hbm.at[gbuf.at[j]], rows, sem).wait()

        @pl.loop(0, K, unroll=8)
        def _edge(k):
            bc = plsc.load_gather(
                sbuf, [jnp.full((LANES,), LANES, jnp.int32) + (j * K + k)])
            for j8 in range(D // LANES):
                sl = pl.ds(j8 * LANES, LANES)
                rows[k, sl] = rows[k, sl] * bc

        pltpu.sync_copy(rows, acc_sp.at[dbuf.at[j]], add=True)

    @pl.loop(0, NSB)
    def _superblock(sb):
        pltpu.sync_copy(g_hbm.at[wid, sb], gbuf)
        pltpu.sync_copy(dst_hbm.at[wid, sb], dbuf)
        pltpu.sync_copy(
            scale_hbm.at[pl.ds(wid * EPW + sb * (NB2 * K), NB2 * K)],
            sbuf.at[pl.ds(LANES, NB2 * K)])

        start_gather(0, rows_a, sem_a)

        @pl.loop(0, (NB2 - 1) // 2)
        def _pair(i):
            j = i * 2
            start_gather(j + 1, rows_b, sem_b)
            finish_block(j, rows_a, sem_a)
            start_gather(j + 2, rows_a, sem_a)
            finish_block(j + 1, rows_b, sem_b)

        finish_block(NB2 - 1, rows_a, sem_a)

    plsc.subcore_barrier()
    for q in range((NCH + NS - 1) // NS):
        ch = s + q * NS

        @pl.when(ch < NCH)
        def _():
            off = ch * K
            pltpu.sync_copy(acc_sp.at[pl.ds(off, K)], rows_a)
            pltpu.sync_copy(rows_a, acc_hbm.at[pl.ds(c * N + off, K)])


BN = 400


def _mm_body(h_ref, w_ref, bf_ref, t_ref, o_ref):
    prod = jnp.dot(h_ref[...], w_ref[...], preferred_element_type=jnp.float32)
    prod = prod + bf_ref[...]
    t_ref[...] = prod[:, :R * D]
    o_ref[...] = prod[:, R * D:]


def _tc_matmul(h, bigw, bfull):
    return pl.pallas_call(
        _mm_body,
        grid=(N // BN,),
        in_specs=[
            pl.BlockSpec((BN, D), lambda i: (i, 0)),
            pl.BlockSpec((D, R * D + D), lambda i: (0, 0)),
            pl.BlockSpec((1, R * D + D), lambda i: (0, 0)),
        ],
        out_specs=[
            pl.BlockSpec((BN, R * D), lambda i: (i, 0)),
            pl.BlockSpec((BN, D), lambda i: (i, 0)),
        ],
        out_shape=[_f32((N, R * D)), _f32((N, D))],
    )(h, bigw, bfull)


def _make_ln_body(layer):
    def body(o_ref, a0_ref, a1_ref, hp_ref, w_ref, b_ref, out_ref):
        v = o_ref[...] + a0_ref[...] + a1_ref[...]
        mu = jnp.mean(v, axis=-1, keepdims=True)
        var = jnp.mean((v - mu) ** 2, axis=-1, keepdims=True)
        y = (v - mu) / jnp.sqrt(var + EPS) * w_ref[...] + b_ref[...]
        y = jnp.maximum(y, 0.0)
        if layer > 0:
            y = y + hp_ref[...]
        out_ref[...] = y
    return body


def _tc_ln(layer, out0, acc0, acc1, h_prev, lnw, lnb):
    return pl.pallas_call(
        _make_ln_body(layer),
        grid=(N // BN,),
        in_specs=[
            pl.BlockSpec((BN, D), lambda i: (i, 0)),
            pl.BlockSpec((BN, D), lambda i: (i, 0)),
            pl.BlockSpec((BN, D), lambda i: (i, 0)),
            pl.BlockSpec((BN, D), lambda i: (i, 0)),
            pl.BlockSpec((1, D), lambda i: (0, 0)),
            pl.BlockSpec((1, D), lambda i: (0, 0)),
        ],
        out_specs=pl.BlockSpec((BN, D), lambda i: (i, 0)),
        out_shape=_f32((N, D)),
    )(out0, acc0, acc1, h_prev, lnw, lnb)


def kernel(x, edge_index, edge_type, emb, W, W_root, b, ln_w, ln_b):
    src = edge_index[0].astype(jnp.int32)
    dst = edge_index[1].astype(jnp.int32)
    ety = edge_type.astype(jnp.int32)
    zcnt = jnp.zeros((N * R,), jnp.float32)
    zacc = jnp.zeros((N, D), jnp.float32)

    g, scale, h = _prep_kernel(src, dst, ety, x.astype(jnp.int32), emb, zcnt)

    for i in range(L_LAYERS):
        bigw = jnp.concatenate(
            [W[i].transpose(1, 0, 2).reshape(D, R * D), W_root[i]], axis=1)
        bfull = jnp.concatenate(
            [jnp.zeros((R * D,), jnp.float32), b[i]]).reshape(1, R * D + D)
        t2d, out0 = _tc_matmul(h, bigw, bfull)
        t = t2d.reshape(N * R, D)
        accf = _agg_kernel(t, g.reshape(NW, NSB, NBLK // NSB, K),
                           dst.reshape(NW, NSB, NBLK // NSB, K), scale, zacc)
        h_new = _tc_ln(i, out0, accf[:N], accf[N:], h,
                       ln_w[i].reshape(1, D), ln_b[i].reshape(1, D))
        h = h_new
    return h


# speedup vs baseline: 23.6634x; 1.3621x over previous
import functools

import jax
import jax.numpy as jnp
from jax import lax
from jax.experimental import pallas as pl
from jax.experimental.pallas import tpu as pltpu
from jax.experimental.pallas import tpu_sc as plsc

N = 10000
E = 320000
D = 128
R = 8
L_LAYERS = 3
EPS = 1e-5

NC = 2
NS = 16
NW = NC * NS
LANES = 16

K = 80
EPW = E // NW
NBLK = EPW // K
NSB = 5
RBLK = (N // K + NW - 1) // NW

_mesh = plsc.VectorSubcoreMesh(core_axis_name="c", subcore_axis_name="s",
                               num_cores=NC, num_subcores=NS)


def _f32(shape):
    return jax.ShapeDtypeStruct(shape, jnp.float32)


def _i32(shape):
    return jax.ShapeDtypeStruct(shape, jnp.int32)


CW = 128
NROW = 640
SBC = 50
SBS = NBLK // NSB
ZCH = NROW // NS


@functools.partial(
    pl.kernel,
    out_type=(_i32((E,)), _f32((E,)), _f32((N, D))),
    mesh=_mesh,
    compiler_params=pltpu.CompilerParams(needs_layout_passes=False),
    scratch_types=[
        pltpu.VMEM_SHARED((NROW, CW), jnp.float32),
        pltpu.VMEM((NROW, CW), jnp.float32),
        pltpu.VMEM((SBC, K), jnp.int32),
        pltpu.VMEM((SBC, K), jnp.int32),
        pltpu.VMEM((SBS, K), jnp.int32),
        pltpu.VMEM((128,), jnp.int32),
        pltpu.VMEM((SBS * K,), jnp.int32),
        pltpu.VMEM((SBS * K,), jnp.float32),
        pltpu.VMEM((ZCH, CW), jnp.float32),
        pltpu.VMEM((K,), jnp.int32),
        pltpu.VMEM((K, D), jnp.float32),
        pltpu.SemaphoreType.DMA,
    ],
)
def _prep_kernel(dstc_hbm, etyc_hbm, src4_hbm, dst4_hbm, ety4_hbm,
                 x_hbm, emb_hbm, zcnt_hbm,
                 g_hbm, scale_hbm, h0_hbm,
                 cnt_sp, cnt_v, mbuf_d, mbuf_t, mbuf_s, ridx, gout, sout,
                 zstage, xbuf, rows, sem):
    c = lax.axis_index("c")
    s = lax.axis_index("s")
    wid = s * NC + c

    pltpu.sync_copy(zcnt_hbm, cnt_v)
    pltpu.sync_copy(zcnt_hbm.at[pl.ds(s * ZCH, ZCH)], zstage)
    pltpu.sync_copy(zstage, cnt_sp.at[pl.ds(s * ZCH, ZCH)])
    plsc.subcore_barrier()

    ones16 = jnp.ones((LANES,), jnp.float32)

    @pl.loop(0, E // NS // (SBC * K))
    def _csuper(u):
        pltpu.sync_copy(dstc_hbm.at[s, u], mbuf_d)
        pltpu.sync_copy(etyc_hbm.at[s, u], mbuf_t)

        @pl.loop(0, SBC)
        def _cblk(j):
            for i in range(K // LANES):
                sl = pl.ds(i * LANES, LANES)
                cidx = mbuf_d[j, sl] * R + mbuf_t[j, sl]
                row = lax.shift_right_logical(cidx, 7)
                col = jnp.bitwise_and(cidx, CW - 1)
                plsc.addupdate_scatter(cnt_v, [row, col], ones16)

    @pl.loop(0, NROW // 128)
    def _reduce(q):
        base = pl.multiple_of(q * 128, 128)
        for i in range(128 // LANES):
            ridx[pl.ds(i * LANES, LANES)] = (
                lax.iota(jnp.int32, LANES) + (base + i * LANES))
        pltpu.sync_copy(cnt_v.at[pl.ds(base, 128)], cnt_sp.at[ridx], add=True)

    plsc.subcore_barrier()
    pltpu.sync_copy(cnt_sp, cnt_v)

    @pl.loop(0, NSB)
    def _ssuper(u):
        pltpu.sync_copy(dst4_hbm.at[wid, u], mbuf_d.at[pl.ds(0, SBS)])
        pltpu.sync_copy(ety4_hbm.at[wid, u], mbuf_t.at[pl.ds(0, SBS)])
        pltpu.sync_copy(src4_hbm.at[wid, u], mbuf_s)

        @pl.loop(0, SBS)
        def _sblk(j):
            for i in range(K // LANES):
                sl = pl.ds(i * LANES, LANES)
                t16 = mbuf_t[j, sl]
                cidx = mbuf_d[j, sl] * R + t16
                row = lax.shift_right_logical(cidx, 7)
                col = jnp.bitwise_and(cidx, CW - 1)
                cv = plsc.load_gather(cnt_v, [row, col])
                osl = pl.ds(j * K + i * LANES, LANES)
                sout[osl] = 1.0 / jnp.maximum(cv, 1.0)
                gout[osl] = mbuf_s[j, sl] * R + t16

        off = wid * EPW + u * (SBS * K)
        pltpu.sync_copy(gout, g_hbm.at[pl.ds(off, SBS * K)])
        pltpu.sync_copy(sout, scale_hbm.at[pl.ds(off, SBS * K)])

    for jj in range(RBLK):
        blk = wid + jj * NW

        @pl.when(blk < N // K)
        def _():
            off = blk * K
            pltpu.sync_copy(x_hbm.at[pl.ds(off, K)], xbuf)
            pltpu.async_copy(emb_hbm.at[xbuf], rows, sem).wait()
            pltpu.sync_copy(rows, h0_hbm.at[pl.ds(off, K)])


@functools.partial(
    pl.kernel,
    out_type=_f32((NC * N, D)),
    mesh=_mesh,
    compiler_params=pltpu.CompilerParams(needs_layout_passes=False),
    scratch_types=[
        pltpu.VMEM_SHARED((N, D), jnp.float32),
        pltpu.VMEM((NBLK // NSB, K), jnp.int32),
        pltpu.VMEM((NBLK // NSB, K), jnp.int32),
        pltpu.VMEM(((NBLK // NSB) * K + LANES,), jnp.float32),
        pltpu.VMEM((K, D), jnp.float32),
        pltpu.VMEM((K, D), jnp.float32),
        pltpu.SemaphoreType.DMA,
        pltpu.SemaphoreType.DMA,
    ],
)
def _agg_kernel(t_hbm, g_hbm, dst_hbm, scale_hbm, zacc_hbm, acc_hbm,
                acc_sp, gbuf, dbuf, sbuf, rows_a, rows_b, sem_a, sem_b):
    c = lax.axis_index("c")
    s = lax.axis_index("s")
    wid = s * NC + c

    NCH = N // K
    for q in range((NCH + NS - 1) // NS):
        ch = s + q * NS

        @pl.when(ch < NCH)
        def _():
            off = ch * K
            pltpu.sync_copy(zacc_hbm.at[pl.ds(off, K)], rows_a)
            pltpu.sync_copy(rows_a, acc_sp.at[pl.ds(off, K)])

    plsc.subcore_barrier()

    NB2 = NBLK // NSB

    def start_gather(j, rows, sem):
        pltpu.async_copy(t_hbm.at[gbuf.at[j]], rows, sem)

    def finish_block(j, rows, sem):
        pltpu.make_async_copy(t_hbm.at[gbuf.at[j]], rows, sem).wait()

        @pl.loop(0, K, unroll=8)
        def _edge(k):
            bc = plsc.load_gather(
                sbuf, [jnp.full((LANES,), LANES, jnp.int32) + (j * K + k)])
            for j8 in range(D // LANES):
                sl = pl.ds(j8 * LANES, LANES)
                rows[k, sl] = rows[k, sl] * bc

        pltpu.sync_copy(rows, acc_sp.at[dbuf.at[j]], add=True)

    @pl.loop(0, NSB)
    def _superblock(sb):
        pltpu.sync_copy(g_hbm.at[wid, sb], gbuf)
        pltpu.sync_copy(dst_hbm.at[wid, sb], dbuf)
        pltpu.sync_copy(
            scale_hbm.at[pl.ds(wid * EPW + sb * (NB2 * K), NB2 * K)],
            sbuf.at[pl.ds(LANES, NB2 * K)])

        start_gather(0, rows_a, sem_a)

        @pl.loop(0, (NB2 - 1) // 2)
        def _pair(i):
            j = i * 2
            start_gather(j + 1, rows_b, sem_b)
            finish_block(j, rows_a, sem_a)
            start_gather(j + 2, rows_a, sem_a)
            finish_block(j + 1, rows_b, sem_b)

        finish_block(NB2 - 1, rows_a, sem_a)

    plsc.subcore_barrier()
    for q in range((NCH + NS - 1) // NS):
        ch = s + q * NS

        @pl.when(ch < NCH)
        def _():
            off = ch * K
            pltpu.sync_copy(acc_sp.at[pl.ds(off, K)], rows_a)
            pltpu.sync_copy(rows_a, acc_hbm.at[pl.ds(c * N + off, K)])


BN = 400


def _mm_body(h_ref, w_ref, bf_ref, t_ref, o_ref):
    prod = jnp.dot(h_ref[...], w_ref[...], preferred_element_type=jnp.float32)
    prod = prod + bf_ref[...]
    t_ref[...] = prod[:, :R * D]
    o_ref[...] = prod[:, R * D:]


def _tc_matmul(h, bigw, bfull):
    return pl.pallas_call(
        _mm_body,
        grid=(N // BN,),
        in_specs=[
            pl.BlockSpec((BN, D), lambda i: (i, 0)),
            pl.BlockSpec((D, R * D + D), lambda i: (0, 0)),
            pl.BlockSpec((1, R * D + D), lambda i: (0, 0)),
        ],
        out_specs=[
            pl.BlockSpec((BN, R * D), lambda i: (i, 0)),
            pl.BlockSpec((BN, D), lambda i: (i, 0)),
        ],
        out_shape=[_f32((N, R * D)), _f32((N, D))],
    )(h, bigw, bfull)


def _make_ln_body(layer):
    def body(o_ref, a0_ref, a1_ref, hp_ref, w_ref, b_ref, out_ref):
        v = o_ref[...] + a0_ref[...] + a1_ref[...]
        mu = jnp.mean(v, axis=-1, keepdims=True)
        var = jnp.mean((v - mu) ** 2, axis=-1, keepdims=True)
        y = (v - mu) / jnp.sqrt(var + EPS) * w_ref[...] + b_ref[...]
        y = jnp.maximum(y, 0.0)
        if layer > 0:
            y = y + hp_ref[...]
        out_ref[...] = y
    return body


def _tc_ln(layer, out0, acc0, acc1, h_prev, lnw, lnb):
    return pl.pallas_call(
        _make_ln_body(layer),
        grid=(N // BN,),
        in_specs=[
            pl.BlockSpec((BN, D), lambda i: (i, 0)),
            pl.BlockSpec((BN, D), lambda i: (i, 0)),
            pl.BlockSpec((BN, D), lambda i: (i, 0)),
            pl.BlockSpec((BN, D), lambda i: (i, 0)),
            pl.BlockSpec((1, D), lambda i: (0, 0)),
            pl.BlockSpec((1, D), lambda i: (0, 0)),
        ],
        out_specs=pl.BlockSpec((BN, D), lambda i: (i, 0)),
        out_shape=_f32((N, D)),
    )(out0, acc0, acc1, h_prev, lnw, lnb)


def kernel(x, edge_index, edge_type, emb, W, W_root, b, ln_w, ln_b):
    src = edge_index[0].astype(jnp.int32)
    dst = edge_index[1].astype(jnp.int32)
    ety = edge_type.astype(jnp.int32)
    zcnt = jnp.zeros((NROW, CW), jnp.float32)
    zacc = jnp.zeros((N, D), jnp.float32)

    shc = (NS, E // NS // (SBC * K), SBC, K)
    shs = (NW, NSB, SBS, K)
    g, scale, h = _prep_kernel(
        dst.reshape(shc), ety.reshape(shc),
        src.reshape(shs), dst.reshape(shs), ety.reshape(shs),
        x.astype(jnp.int32), emb, zcnt)

    for i in range(L_LAYERS):
        bigw = jnp.concatenate(
            [W[i].transpose(1, 0, 2).reshape(D, R * D), W_root[i]], axis=1)
        bfull = jnp.concatenate(
            [jnp.zeros((R * D,), jnp.float32), b[i]]).reshape(1, R * D + D)
        t2d, out0 = _tc_matmul(h, bigw, bfull)
        t = t2d.reshape(N * R, D)
        accf = _agg_kernel(t, g.reshape(NW, NSB, NBLK // NSB, K),
                           dst.reshape(NW, NSB, NBLK // NSB, K), scale, zacc)
        h_new = _tc_ln(i, out0, accf[:N], accf[N:], h,
                       ln_w[i].reshape(1, D), ln_b[i].reshape(1, D))
        h = h_new
    return h


# speedup vs baseline: 25.6471x; 1.0838x over previous
import functools

import jax
import jax.numpy as jnp
from jax import lax
from jax.experimental import pallas as pl
from jax.experimental.pallas import tpu as pltpu
from jax.experimental.pallas import tpu_sc as plsc

N = 10000
E = 320000
D = 128
R = 8
L_LAYERS = 3
EPS = 1e-5

NC = 2
NS = 16
NW = NC * NS
LANES = 16

K = 80
EPW = E // NW
NBLK = EPW // K
NSB = 5
RBLK = (N // K + NW - 1) // NW

_mesh = plsc.VectorSubcoreMesh(core_axis_name="c", subcore_axis_name="s",
                               num_cores=NC, num_subcores=NS)


def _f32(shape):
    return jax.ShapeDtypeStruct(shape, jnp.float32)


def _i32(shape):
    return jax.ShapeDtypeStruct(shape, jnp.int32)


CW = 128
NROW = 640
SBC = 50
SBS = NBLK // NSB
ZCH = NROW // NS


@functools.partial(
    pl.kernel,
    out_type=(_i32((E,)), _f32((E,)), _f32((N, D))),
    mesh=_mesh,
    compiler_params=pltpu.CompilerParams(needs_layout_passes=False),
    scratch_types=[
        pltpu.VMEM_SHARED((NROW, CW), jnp.float32),
        pltpu.VMEM((NROW, CW), jnp.float32),
        pltpu.VMEM((SBC, K), jnp.int32),
        pltpu.VMEM((SBC, K), jnp.int32),
        pltpu.VMEM((SBS, K), jnp.int32),
        pltpu.VMEM((128,), jnp.int32),
        pltpu.VMEM((SBS * K,), jnp.int32),
        pltpu.VMEM((SBS * K,), jnp.float32),
        pltpu.VMEM((ZCH, CW), jnp.float32),
        pltpu.VMEM((K,), jnp.int32),
        pltpu.VMEM((K, D), jnp.float32),
        pltpu.SemaphoreType.DMA,
    ],
)
def _prep_kernel(dstc_hbm, etyc_hbm, src4_hbm, dst4_hbm, ety4_hbm,
                 x_hbm, emb_hbm, zcnt_hbm,
                 g_hbm, scale_hbm, h0_hbm,
                 cnt_sp, cnt_v, mbuf_d, mbuf_t, mbuf_s, ridx, gout, sout,
                 zstage, xbuf, rows, sem):
    c = lax.axis_index("c")
    s = lax.axis_index("s")
    wid = s * NC + c

    pltpu.sync_copy(zcnt_hbm, cnt_v)
    pltpu.sync_copy(zcnt_hbm.at[pl.ds(s * ZCH, ZCH)], zstage)
    pltpu.sync_copy(zstage, cnt_sp.at[pl.ds(s * ZCH, ZCH)])
    plsc.subcore_barrier()

    ones16 = jnp.ones((LANES,), jnp.float32)

    @pl.loop(0, E // NS // (SBC * K))
    def _csuper(u):
        pltpu.sync_copy(dstc_hbm.at[s, u], mbuf_d)
        pltpu.sync_copy(etyc_hbm.at[s, u], mbuf_t)

        @pl.loop(0, SBC)
        def _cblk(j):
            for i in range(K // LANES):
                sl = pl.ds(i * LANES, LANES)
                cidx = mbuf_d[j, sl] * R + mbuf_t[j, sl]
                row = lax.shift_right_logical(cidx, 7)
                col = jnp.bitwise_and(cidx, CW - 1)
                plsc.addupdate_scatter(cnt_v, [row, col], ones16)

    @pl.loop(0, NROW // 128)
    def _reduce(q):
        base = pl.multiple_of(q * 128, 128)
        for i in range(128 // LANES):
            ridx[pl.ds(i * LANES, LANES)] = (
                lax.iota(jnp.int32, LANES) + (base + i * LANES))
        pltpu.sync_copy(cnt_v.at[pl.ds(base, 128)], cnt_sp.at[ridx], add=True)

    plsc.subcore_barrier()
    pltpu.sync_copy(cnt_sp, cnt_v)

    @pl.loop(0, NSB)
    def _ssuper(u):
        pltpu.sync_copy(dst4_hbm.at[wid, u], mbuf_d.at[pl.ds(0, SBS)])
        pltpu.sync_copy(ety4_hbm.at[wid, u], mbuf_t.at[pl.ds(0, SBS)])
        pltpu.sync_copy(src4_hbm.at[wid, u], mbuf_s)

        @pl.loop(0, SBS)
        def _sblk(j):
            for i in range(K // LANES):
                sl = pl.ds(i * LANES, LANES)
                t16 = mbuf_t[j, sl]
                cidx = mbuf_d[j, sl] * R + t16
                row = lax.shift_right_logical(cidx, 7)
                col = jnp.bitwise_and(cidx, CW - 1)
                cv = plsc.load_gather(cnt_v, [row, col])
                osl = pl.ds(j * K + i * LANES, LANES)
                sout[osl] = 1.0 / jnp.maximum(cv, 1.0)
                gout[osl] = mbuf_s[j, sl] * R + t16

        off = wid * EPW + u * (SBS * K)
        pltpu.sync_copy(gout, g_hbm.at[pl.ds(off, SBS * K)])
        pltpu.sync_copy(sout, scale_hbm.at[pl.ds(off, SBS * K)])

    for jj in range(RBLK):
        blk = wid + jj * NW

        @pl.when(blk < N // K)
        def _():
            off = blk * K
            pltpu.sync_copy(x_hbm.at[pl.ds(off, K)], xbuf)
            pltpu.async_copy(emb_hbm.at[xbuf], rows, sem).wait()
            pltpu.sync_copy(rows, h0_hbm.at[pl.ds(off, K)])


@functools.partial(
    pl.kernel,
    out_type=_f32((NC * N, D)),
    mesh=_mesh,
    compiler_params=pltpu.CompilerParams(needs_layout_passes=False),
    scratch_types=[
        pltpu.VMEM_SHARED((N, D), jnp.float32),
        pltpu.VMEM((NBLK // NSB, K), jnp.int32),
        pltpu.VMEM((NBLK // NSB, K), jnp.int32),
        pltpu.VMEM(((NBLK // NSB) * K + LANES,), jnp.float32),
        pltpu.VMEM((K, D), jnp.float32),
        pltpu.VMEM((K, D), jnp.float32),
        pltpu.VMEM((K, D), jnp.float32),
        pltpu.SemaphoreType.DMA,
        pltpu.SemaphoreType.DMA,
        pltpu.SemaphoreType.DMA,
        pltpu.SemaphoreType.DMA,
        pltpu.SemaphoreType.DMA,
        pltpu.SemaphoreType.DMA,
    ],
)
def _agg_kernel(t_hbm, g_hbm, dst_hbm, scale_hbm, zacc_hbm, acc_hbm,
                acc_sp, gbuf, dbuf, sbuf, rows_a, rows_b, rows_c,
                sem_a, sem_b, sem_c, ssem_a, ssem_b, ssem_c):
    c = lax.axis_index("c")
    s = lax.axis_index("s")
    wid = s * NC + c

    NCH = N // K
    for q in range((NCH + NS - 1) // NS):
        ch = s + q * NS

        @pl.when(ch < NCH)
        def _():
            off = ch * K
            pltpu.sync_copy(zacc_hbm.at[pl.ds(off, K)], rows_a)
            pltpu.sync_copy(rows_a, acc_sp.at[pl.ds(off, K)])

    plsc.subcore_barrier()

    NB2 = NBLK // NSB
    bufs = ((rows_a, sem_a, ssem_a), (rows_b, sem_b, ssem_b),
            (rows_c, sem_c, ssem_c))

    def start_gather(j, x):
        pltpu.async_copy(t_hbm.at[gbuf.at[j]], bufs[x][0], bufs[x][1])

    def scale_scatter(j, x):
        rows, sem, ssem = bufs[x]
        pltpu.make_async_copy(t_hbm.at[gbuf.at[j]], rows, sem).wait()

        @pl.loop(0, K, unroll=8)
        def _edge(k):
            bc = plsc.load_gather(
                sbuf, [jnp.full((LANES,), LANES, jnp.int32) + (j * K + k)])
            for j8 in range(D // LANES):
                sl = pl.ds(j8 * LANES, LANES)
                rows[k, sl] = rows[k, sl] * bc

        pltpu.async_copy(rows, acc_sp.at[dbuf.at[j]], ssem, add=True)

    def wait_scatter(x):
        rows, _, ssem = bufs[x]
        pltpu.make_async_copy(rows, acc_sp.at[dbuf.at[0]], ssem).wait()

    @pl.loop(0, NSB)
    def _superblock(sb):
        pltpu.sync_copy(g_hbm.at[wid, sb], gbuf)
        pltpu.sync_copy(dst_hbm.at[wid, sb], dbuf)
        pltpu.sync_copy(
            scale_hbm.at[pl.ds(wid * EPW + sb * (NB2 * K), NB2 * K)],
            sbuf.at[pl.ds(LANES, NB2 * K)])

        start_gather(0, 0)
        start_gather(1, 1)
        scale_scatter(0, 0)
        start_gather(2, 2)

        @pl.loop(0, (NB2 - 1) // 3)
        def _trio(i):
            for t in range(3):
                b = i * 3 + 1 + t
                x = (1 + t) % 3
                xp = t % 3
                scale_scatter(b, x)
                wait_scatter(xp)
                if t == 0:
                    start_gather(b + 2, xp)
                else:
                    @pl.when(b + 2 <= NB2 - 1)
                    def _():
                        start_gather(b + 2, xp)

        wait_scatter((NB2 - 1) % 3)

    plsc.subcore_barrier()
    for q in range((NCH + NS - 1) // NS):
        ch = s + q * NS

        @pl.when(ch < NCH)
        def _():
            off = ch * K
            pltpu.sync_copy(acc_sp.at[pl.ds(off, K)], rows_a)
            pltpu.sync_copy(rows_a, acc_hbm.at[pl.ds(c * N + off, K)])


BN = 400


def _mm_body(h_ref, w_ref, bf_ref, t_ref, o_ref):
    prod = jnp.dot(h_ref[...], w_ref[...], preferred_element_type=jnp.float32)
    prod = prod + bf_ref[...]
    t_ref[...] = prod[:, :R * D]
    o_ref[...] = prod[:, R * D:]


def _tc_matmul(h, bigw, bfull):
    return pl.pallas_call(
        _mm_body,
        grid=(N // BN,),
        in_specs=[
            pl.BlockSpec((BN, D), lambda i: (i, 0)),
            pl.BlockSpec((D, R * D + D), lambda i: (0, 0)),
            pl.BlockSpec((1, R * D + D), lambda i: (0, 0)),
        ],
        out_specs=[
            pl.BlockSpec((BN, R * D), lambda i: (i, 0)),
            pl.BlockSpec((BN, D), lambda i: (i, 0)),
        ],
        out_shape=[_f32((N, R * D)), _f32((N, D))],
    )(h, bigw, bfull)


def _make_ln_body(layer):
    def body(o_ref, a0_ref, a1_ref, hp_ref, w_ref, b_ref, out_ref):
        v = o_ref[...] + a0_ref[...] + a1_ref[...]
        mu = jnp.mean(v, axis=-1, keepdims=True)
        var = jnp.mean((v - mu) ** 2, axis=-1, keepdims=True)
        y = (v - mu) / jnp.sqrt(var + EPS) * w_ref[...] + b_ref[...]
        y = jnp.maximum(y, 0.0)
        if layer > 0:
            y = y + hp_ref[...]
        out_ref[...] = y
    return body


def _tc_ln(layer, out0, acc0, acc1, h_prev, lnw, lnb):
    return pl.pallas_call(
        _make_ln_body(layer),
        grid=(N // BN,),
        in_specs=[
            pl.BlockSpec((BN, D), lambda i: (i, 0)),
            pl.BlockSpec((BN, D), lambda i: (i, 0)),
            pl.BlockSpec((BN, D), lambda i: (i, 0)),
            pl.BlockSpec((BN, D), lambda i: (i, 0)),
            pl.BlockSpec((1, D), lambda i: (0, 0)),
            pl.BlockSpec((1, D), lambda i: (0, 0)),
        ],
        out_specs=pl.BlockSpec((BN, D), lambda i: (i, 0)),
        out_shape=_f32((N, D)),
    )(out0, acc0, acc1, h_prev, lnw, lnb)


def kernel(x, edge_index, edge_type, emb, W, W_root, b, ln_w, ln_b):
    src = edge_index[0].astype(jnp.int32)
    dst = edge_index[1].astype(jnp.int32)
    ety = edge_type.astype(jnp.int32)
    zcnt = jnp.zeros((NROW, CW), jnp.float32)
    zacc = jnp.zeros((N, D), jnp.float32)

    shc = (NS, E // NS // (SBC * K), SBC, K)
    shs = (NW, NSB, SBS, K)
    g, scale, h = _prep_kernel(
        dst.reshape(shc), ety.reshape(shc),
        src.reshape(shs), dst.reshape(shs), ety.reshape(shs),
        x.astype(jnp.int32), emb, zcnt)

    for i in range(L_LAYERS):
        bigw = jnp.concatenate(
            [W[i].transpose(1, 0, 2).reshape(D, R * D), W_root[i]], axis=1)
        bfull = jnp.concatenate(
            [jnp.zeros((R * D,), jnp.float32), b[i]]).reshape(1, R * D + D)
        t2d, out0 = _tc_matmul(h, bigw, bfull)
        t = t2d.reshape(N * R, D)
        accf = _agg_kernel(t, g.reshape(NW, NSB, NBLK // NSB, K),
                           dst.reshape(NW, NSB, NBLK // NSB, K), scale, zacc)
        h_new = _tc_ln(i, out0, accf[:N], accf[N:], h,
                       ln_w[i].reshape(1, D), ln_b[i].reshape(1, D))
        h = h_new
    return h


# speedup vs baseline: 26.4565x; 1.0316x over previous
import functools

import jax
import jax.numpy as jnp
from jax import lax
from jax.experimental import pallas as pl
from jax.experimental.pallas import tpu as pltpu
from jax.experimental.pallas import tpu_sc as plsc

N = 10000
E = 320000
D = 128
R = 8
L_LAYERS = 3
EPS = 1e-5

NC = 2
NS = 16
NW = NC * NS
LANES = 16

K = 80
EPW = E // NW
NBLK = EPW // K
NSB = 5
RBLK = (N // K + NW - 1) // NW

_mesh = plsc.VectorSubcoreMesh(core_axis_name="c", subcore_axis_name="s",
                               num_cores=NC, num_subcores=NS)


def _f32(shape):
    return jax.ShapeDtypeStruct(shape, jnp.float32)


def _i32(shape):
    return jax.ShapeDtypeStruct(shape, jnp.int32)


CW = 128
NROW = 640
SBC = 50
SBS = NBLK // NSB
ZCH = NROW // NS


@functools.partial(
    pl.kernel,
    out_type=(_i32((E,)), _f32((E,)), _f32((N, D))),
    mesh=_mesh,
    compiler_params=pltpu.CompilerParams(needs_layout_passes=False),
    scratch_types=[
        pltpu.VMEM_SHARED((NROW, CW), jnp.float32),
        pltpu.VMEM((NROW, CW), jnp.float32),
        pltpu.VMEM((SBC, K), jnp.int32),
        pltpu.VMEM((SBC, K), jnp.int32),
        pltpu.VMEM((SBS, K), jnp.int32),
        pltpu.VMEM((128,), jnp.int32),
        pltpu.VMEM((SBS * K,), jnp.int32),
        pltpu.VMEM((SBS * K,), jnp.float32),
        pltpu.VMEM((ZCH, CW), jnp.float32),
        pltpu.VMEM((K,), jnp.int32),
        pltpu.VMEM((K, D), jnp.float32),
        pltpu.SemaphoreType.DMA,
    ],
)
def _prep_kernel(dstc_hbm, etyc_hbm, src4_hbm, dst4_hbm, ety4_hbm,
                 x_hbm, emb_hbm, zcnt_hbm,
                 g_hbm, scale_hbm, h0_hbm,
                 cnt_sp, cnt_v, mbuf_d, mbuf_t, mbuf_s, ridx, gout, sout,
                 zstage, xbuf, rows, sem):
    c = lax.axis_index("c")
    s = lax.axis_index("s")
    wid = s * NC + c

    pltpu.sync_copy(zcnt_hbm, cnt_v)
    pltpu.sync_copy(zcnt_hbm.at[pl.ds(s * ZCH, ZCH)], zstage)
    pltpu.sync_copy(zstage, cnt_sp.at[pl.ds(s * ZCH, ZCH)])
    plsc.subcore_barrier()

    ones16 = jnp.ones((LANES,), jnp.float32)

    @pl.loop(0, E // NS // (SBC * K))
    def _csuper(u):
        pltpu.sync_copy(dstc_hbm.at[s, u], mbuf_d)
        pltpu.sync_copy(etyc_hbm.at[s, u], mbuf_t)

        @pl.loop(0, SBC)
        def _cblk(j):
            for i in range(K // LANES):
                sl = pl.ds(i * LANES, LANES)
                cidx = mbuf_d[j, sl] * R + mbuf_t[j, sl]
                row = lax.shift_right_logical(cidx, 7)
                col = jnp.bitwise_and(cidx, CW - 1)
                plsc.addupdate_scatter(cnt_v, [row, col], ones16)

    @pl.loop(0, NROW // 128)
    def _reduce(q):
        base = pl.multiple_of(q * 128, 128)
        for i in range(128 // LANES):
            ridx[pl.ds(i * LANES, LANES)] = (
                lax.iota(jnp.int32, LANES) + (base + i * LANES))
        pltpu.sync_copy(cnt_v.at[pl.ds(base, 128)], cnt_sp.at[ridx], add=True)

    plsc.subcore_barrier()
    pltpu.sync_copy(cnt_sp, cnt_v)

    @pl.loop(0, NSB)
    def _ssuper(u):
        pltpu.sync_copy(dst4_hbm.at[wid, u], mbuf_d.at[pl.ds(0, SBS)])
        pltpu.sync_copy(ety4_hbm.at[wid, u], mbuf_t.at[pl.ds(0, SBS)])
        pltpu.sync_copy(src4_hbm.at[wid, u], mbuf_s)

        @pl.loop(0, SBS)
        def _sblk(j):
            for i in range(K // LANES):
                sl = pl.ds(i * LANES, LANES)
                t16 = mbuf_t[j, sl]
                cidx = mbuf_d[j, sl] * R + t16
                row = lax.shift_right_logical(cidx, 7)
                col = jnp.bitwise_and(cidx, CW - 1)
                cv = plsc.load_gather(cnt_v, [row, col])
                osl = pl.ds(j * K + i * LANES, LANES)
                sout[osl] = 1.0 / jnp.maximum(cv, 1.0)
                gout[osl] = mbuf_s[j, sl] * R + t16

        off = wid * EPW + u * (SBS * K)
        pltpu.sync_copy(gout, g_hbm.at[pl.ds(off, SBS * K)])
        pltpu.sync_copy(sout, scale_hbm.at[pl.ds(off, SBS * K)])

    for jj in range(RBLK):
        blk = wid + jj * NW

        @pl.when(blk < N // K)
        def _():
            off = blk * K
            pltpu.sync_copy(x_hbm.at[pl.ds(off, K)], xbuf)
            pltpu.async_copy(emb_hbm.at[xbuf], rows, sem).wait()
            pltpu.sync_copy(rows, h0_hbm.at[pl.ds(off, K)])


@functools.partial(
    pl.kernel,
    out_type=_f32((NC * N, D)),
    mesh=_mesh,
    compiler_params=pltpu.CompilerParams(needs_layout_passes=False),
    scratch_types=[
        pltpu.VMEM_SHARED((N, D), jnp.float32),
        pltpu.VMEM((NBLK // NSB, K), jnp.int32),
        pltpu.VMEM((NBLK // NSB, K), jnp.int32),
        pltpu.VMEM(((NBLK // NSB) * K + LANES,), jnp.float32),
        pltpu.VMEM((K, D), jnp.float32),
        pltpu.VMEM((K, D), jnp.float32),
        pltpu.VMEM((K, D), jnp.float32),
        pltpu.SemaphoreType.DMA,
        pltpu.SemaphoreType.DMA,
        pltpu.SemaphoreType.DMA,
        pltpu.SemaphoreType.DMA,
        pltpu.SemaphoreType.DMA,
        pltpu.SemaphoreType.DMA,
    ],
)
def _agg_kernel(t_hbm, g_hbm, dst_hbm, scale_hbm, zacc_hbm, acc_hbm,
                acc_sp, gbuf, dbuf, sbuf, rows_a, rows_b, rows_c,
                sem_a, sem_b, sem_c, ssem_a, ssem_b, ssem_c):
    c = lax.axis_index("c")
    s = lax.axis_index("s")
    wid = s * NC + c

    NCH = N // K
    for q in range((NCH + NS - 1) // NS):
        ch = s + q * NS

        @pl.when(ch < NCH)
        def _():
            off = ch * K
            pltpu.sync_copy(zacc_hbm.at[pl.ds(off, K)], rows_a)
            pltpu.sync_copy(rows_a, acc_sp.at[pl.ds(off, K)])

    plsc.subcore_barrier()

    NB2 = NBLK // NSB
    bufs = ((rows_a, sem_a, ssem_a), (rows_b, sem_b, ssem_b),
            (rows_c, sem_c, ssem_c))

    def start_gather(j, x):
        pltpu.async_copy(t_hbm.at[gbuf.at[j]], bufs[x][0], bufs[x][1])

    def scale_scatter(j, x):
        rows, sem, ssem = bufs[x]
        pltpu.make_async_copy(t_hbm.at[gbuf.at[j]], rows, sem).wait()

        @pl.loop(0, K, unroll=8)
        def _edge(k):
            bc = plsc.load_gather(
                sbuf, [jnp.full((LANES,), LANES, jnp.int32) + (j * K + k)])
            for j8 in range(D // LANES):
                sl = pl.ds(j8 * LANES, LANES)
                rows[k, sl] = rows[k, sl] * bc

        pltpu.async_copy(rows, acc_sp.at[dbuf.at[j]], ssem, add=True)

    def wait_scatter(x):
        rows, _, ssem = bufs[x]
        pltpu.make_async_copy(rows, acc_sp.at[dbuf.at[0]], ssem).wait()

    @pl.loop(0, NSB)
    def _superblock(sb):
        pltpu.sync_copy(g_hbm.at[wid, sb], gbuf)
        pltpu.sync_copy(dst_hbm.at[wid, sb], dbuf)
        pltpu.sync_copy(
            scale_hbm.at[pl.ds(wid * EPW + sb * (NB2 * K), NB2 * K)],
            sbuf.at[pl.ds(LANES, NB2 * K)])

        start_gather(0, 0)
        start_gather(1, 1)
        scale_scatter(0, 0)
        start_gather(2, 2)

        @pl.loop(0, (NB2 - 1) // 3)
        def _trio(i):
            for t in range(3):
                b = i * 3 + 1 + t
                x = (1 + t) % 3
                xp = t % 3
                scale_scatter(b, x)
                wait_scatter(xp)
                if t == 0:
                    start_gather(b + 2, xp)
                else:
                    @pl.when(b + 2 <= NB2 - 1)
                    def _():
                        start_gather(b + 2, xp)

        wait_scatter((NB2 - 1) % 3)

    plsc.subcore_barrier()
    for q in range((NCH + NS - 1) // NS):
        ch = s + q * NS

        @pl.when(ch < NCH)
        def _():
            off = ch * K
            pltpu.sync_copy(acc_sp.at[pl.ds(off, K)], rows_a)
            pltpu.sync_copy(rows_a, acc_hbm.at[pl.ds(c * N + off, K)])


BN = 400


def _mm_body(h_ref, w_ref, bf_ref, t_ref, o_ref):
    prod = jnp.dot(h_ref[...], w_ref[...], preferred_element_type=jnp.float32)
    prod = prod + bf_ref[...]
    t_ref[...] = prod[:, :R * D]
    o_ref[...] = prod[:, R * D:]


def _tc_matmul(h, bigw, bfull):
    return pl.pallas_call(
        _mm_body,
        grid=(N // BN,),
        in_specs=[
            pl.BlockSpec((BN, D), lambda i: (i, 0)),
            pl.BlockSpec((D, R * D + D), lambda i: (0, 0)),
            pl.BlockSpec((1, R * D + D), lambda i: (0, 0)),
        ],
        out_specs=[
            pl.BlockSpec((BN, R * D), lambda i: (i, 0)),
            pl.BlockSpec((BN, D), lambda i: (i, 0)),
        ],
        out_shape=[_f32((N, R * D)), _f32((N, D))],
    )(h, bigw, bfull)


def _ln_value(layer, o_ref, a0_ref, a1_ref, hp_ref, w_ref, b_ref):
    v = o_ref[...] + a0_ref[...] + a1_ref[...]
    mu = jnp.mean(v, axis=-1, keepdims=True)
    var = jnp.mean((v - mu) ** 2, axis=-1, keepdims=True)
    y = (v - mu) / jnp.sqrt(var + EPS) * w_ref[...] + b_ref[...]
    y = jnp.maximum(y, 0.0)
    if layer > 0:
        y = y + hp_ref[...]
    return y


def _make_ln_body(layer):
    def body(o_ref, a0_ref, a1_ref, hp_ref, w_ref, b_ref, out_ref):
        out_ref[...] = _ln_value(layer, o_ref, a0_ref, a1_ref, hp_ref,
                                 w_ref, b_ref)
    return body


def _tc_ln(layer, out0, acc0, acc1, h_prev, lnw, lnb):
    return pl.pallas_call(
        _make_ln_body(layer),
        grid=(N // BN,),
        in_specs=[
            pl.BlockSpec((BN, D), lambda i: (i, 0)),
            pl.BlockSpec((BN, D), lambda i: (i, 0)),
            pl.BlockSpec((BN, D), lambda i: (i, 0)),
            pl.BlockSpec((BN, D), lambda i: (i, 0)),
            pl.BlockSpec((1, D), lambda i: (0, 0)),
            pl.BlockSpec((1, D), lambda i: (0, 0)),
        ],
        out_specs=pl.BlockSpec((BN, D), lambda i: (i, 0)),
        out_shape=_f32((N, D)),
    )(out0, acc0, acc1, h_prev, lnw, lnb)


def _make_ln_mm_body(layer):
    def body(o_ref, a0_ref, a1_ref, hp_ref, w_ref, b_ref, bw_ref, bf_ref,
             t_ref, o2_ref, h_ref):
        y = _ln_value(layer, o_ref, a0_ref, a1_ref, hp_ref, w_ref, b_ref)
        h_ref[...] = y
        prod = jnp.dot(y, bw_ref[...], preferred_element_type=jnp.float32)
        prod = prod + bf_ref[...]
        t_ref[...] = prod[:, :R * D]
        o2_ref[...] = prod[:, R * D:]
    return body


def _tc_ln_mm(layer, out0, acc0, acc1, h_prev, lnw, lnb, bigw, bfull):
    return pl.pallas_call(
        _make_ln_mm_body(layer),
        grid=(N // BN,),
        in_specs=[
            pl.BlockSpec((BN, D), lambda i: (i, 0)),
            pl.BlockSpec((BN, D), lambda i: (i, 0)),
            pl.BlockSpec((BN, D), lambda i: (i, 0)),
            pl.BlockSpec((BN, D), lambda i: (i, 0)),
            pl.BlockSpec((1, D), lambda i: (0, 0)),
            pl.BlockSpec((1, D), lambda i: (0, 0)),
            pl.BlockSpec((D, R * D + D), lambda i: (0, 0)),
            pl.BlockSpec((1, R * D + D), lambda i: (0, 0)),
        ],
        out_specs=[
            pl.BlockSpec((BN, R * D), lambda i: (i, 0)),
            pl.BlockSpec((BN, D), lambda i: (i, 0)),
            pl.BlockSpec((BN, D), lambda i: (i, 0)),
        ],
        out_shape=[_f32((N, R * D)), _f32((N, D)), _f32((N, D))],
    )(out0, acc0, acc1, h_prev, lnw, lnb, bigw, bfull)


def kernel(x, edge_index, edge_type, emb, W, W_root, b, ln_w, ln_b):
    src = edge_index[0].astype(jnp.int32)
    dst = edge_index[1].astype(jnp.int32)
    ety = edge_type.astype(jnp.int32)
    zcnt = jnp.zeros((NROW, CW), jnp.float32)
    zacc = jnp.zeros((N, D), jnp.float32)

    shc = (NS, E // NS // (SBC * K), SBC, K)
    shs = (NW, NSB, SBS, K)
    g, scale, h = _prep_kernel(
        dst.reshape(shc), ety.reshape(shc),
        src.reshape(shs), dst.reshape(shs), ety.reshape(shs),
        x.astype(jnp.int32), emb, zcnt)

    g4 = g.reshape(NW, NSB, NBLK // NSB, K)
    dst4 = dst.reshape(NW, NSB, NBLK // NSB, K)

    def wcat(i):
        bigw = jnp.concatenate(
            [W[i].transpose(1, 0, 2).reshape(D, R * D), W_root[i]], axis=1)
        bfull = jnp.concatenate(
            [jnp.zeros((R * D,), jnp.float32), b[i]]).reshape(1, R * D + D)
        return bigw, bfull

    t2d, out0 = _tc_matmul(h, *wcat(0))
    for i in range(L_LAYERS):
        accf = _agg_kernel(t2d.reshape(N * R, D), g4, dst4, scale, zacc)
        lnw, lnb = ln_w[i].reshape(1, D), ln_b[i].reshape(1, D)
        if i + 1 < L_LAYERS:
            t2d, out0, h = _tc_ln_mm(i, out0, accf[:N], accf[N:], h,
                                     lnw, lnb, *wcat(i + 1))
        else:
            h = _tc_ln(i, out0, accf[:N], accf[N:], h, lnw, lnb)
    return h
